# Initial kernel scaffold; baseline (speedup 1.0000x reference)
import jax
import jax.numpy as jnp
from jax.experimental import pallas as pl


def kernel(x, edge_index, W_in, b_in, Wself0, Wneigh0, b0, Wself1, Wneigh1, b1, W_out, b_out):
    raise NotImplementedError("write your pallas kernel here")


# speedup vs baseline: 10.2463x; 10.2463x over previous
import jax
import jax.numpy as jnp
import numpy as np
from jax import lax
from jax.experimental import pallas as pl
from jax.experimental.pallas import tpu as pltpu
from jax.experimental.pallas import tpu_sc as plsc

N = 10000
E = 320000
F = 128
K = 32

NPAD = 10240
_NC = 2
_NS = 16
_NW = _NC * _NS
_CHUNK = 128
_NBLK = E // _CHUNK
_BLK_PER_W = -(-_NBLK // _NW)
_RPT = NPAD // _NS
_R = 1000
_NB = N // _R


def _seg_body(h_hbm, src_hbm, dst_hbm, s_hbm, deg_hbm,
              acc_sh, deg_sh, zb_v, src_v, dst_v, rows_v, ones_v, gsem):
    cid = lax.axis_index("c")
    sid = lax.axis_index("s")
    wid = sid * _NC + cid

    def zfill(i, carry):
        r = i // 8
        c = (i % 8) * 16
        zb_v[r, pl.ds(c, 16)] = jnp.zeros((16,), jnp.float32)
        return carry
    lax.fori_loop(0, 128 * 8, zfill, 0)

    def ofill(i, carry):
        ones_v[pl.ds(i * 16, 16)] = jnp.ones((16,), jnp.float32)
        return carry
    lax.fori_loop(0, 8, ofill, 0)

    base_row = pl.multiple_of(sid * _RPT, 128)

    def zslab(k, carry):
        off = pl.multiple_of(base_row + k * 128, 128)
        pltpu.sync_copy(zb_v, acc_sh.at[pl.ds(off, 128)])
        pltpu.sync_copy(zb_v.at[0], deg_sh.at[pl.ds(off, 128)])
        return carry
    lax.fori_loop(0, _RPT // 128, zslab, 0)

    plsc.subcore_barrier()

    def step(j, carry):
        blk = wid + _NW * j

        @pl.when(blk < _NBLK)
        def _():
            base = pl.multiple_of(blk * _CHUNK, _CHUNK)
            pltpu.sync_copy(src_hbm.at[pl.ds(base, _CHUNK)], src_v)
            pltpu.sync_copy(dst_hbm.at[pl.ds(base, _CHUNK)], dst_v)
            pltpu.async_copy(h_hbm.at[src_v], rows_v, gsem).wait()
            pltpu.sync_copy(rows_v, acc_sh.at[dst_v], add=True)
            pltpu.sync_copy(ones_v, deg_sh.at[dst_v], add=True)
        return carry
    lax.fori_loop(0, _BLK_PER_W, step, 0)

    plsc.subcore_barrier()

    pltpu.sync_copy(acc_sh.at[pl.ds(base_row, _RPT)],
                    s_hbm.at[cid, pl.ds(base_row, _RPT)])
    pltpu.sync_copy(deg_sh.at[pl.ds(base_row, _RPT)],
                    deg_hbm.at[cid, pl.ds(base_row, _RPT)])


def _segsum(h, src, dst):
    call = pl.kernel(
        _seg_body,
        out_type=[
            jax.ShapeDtypeStruct((_NC, NPAD, F), jnp.float32),
            jax.ShapeDtypeStruct((_NC, NPAD), jnp.float32),
        ],
        mesh=plsc.VectorSubcoreMesh(core_axis_name="c", subcore_axis_name="s"),
        scratch_types=[
            pltpu.VMEM_SHARED((NPAD, F), jnp.float32),
            pltpu.VMEM_SHARED((NPAD,), jnp.float32),
            pltpu.VMEM((128, F), jnp.float32),
            pltpu.VMEM((_CHUNK,), jnp.int32),
            pltpu.VMEM((_CHUNK,), jnp.int32),
            pltpu.VMEM((_CHUNK, F), jnp.float32),
            pltpu.VMEM((_CHUNK,), jnp.float32),
            pltpu.SemaphoreType.DMA,
        ],
    )
    return call(h, src, dst)


_MIN32 = np.int32(-(2 ** 31))


def _maxk_rows(x):
    key = lax.bitcast_convert_type(x, jnp.int32)
    key = key ^ ((key >> 31) & np.int32(0x7FFFFFFF))
    t = jnp.zeros((x.shape[0], 1), jnp.int32)
    for i in range(31, -1, -1):
        bit = _MIN32 if i == 31 else np.int32(1 << i)
        cand = t | bit
        cnt = jnp.sum((key >= (cand ^ _MIN32)).astype(jnp.int32),
                      axis=1, keepdims=True)
        t = jnp.where(cnt >= K, cand, t)
    thr = t ^ _MIN32
    gt = key > thr
    eq = key == thr
    ngt = jnp.sum(gt.astype(jnp.int32), axis=1, keepdims=True)
    quota = (K - ngt).astype(jnp.float32)
    ii = lax.broadcasted_iota(jnp.int32, (F, F), 0)
    jj = lax.broadcasted_iota(jnp.int32, (F, F), 1)
    upper = (ii <= jj).astype(jnp.float32)
    run = lax.dot_general(eq.astype(jnp.float32), upper,
                          (((1,), (0,)), ((), ())),
                          preferred_element_type=jnp.float32)
    keep = gt | (eq & (run <= quota))
    return jnp.where(keep, x, jnp.zeros_like(x))


def _mm(a, w):
    return lax.dot_general(a, w, (((1,), (1,)), ((), ())),
                           preferred_element_type=jnp.float32)


def _tc1_body(x_ref, w_ref, b_ref, o_ref):
    h = _mm(x_ref[...], w_ref[...]) + b_ref[...]
    o_ref[...] = _maxk_rows(h)


def _neigh(s_ref, d_ref):
    d = d_ref[0] + d_ref[1]
    return (s_ref[0] + s_ref[1]) * (1.0 / jnp.maximum(d, 1.0))


def _tc2_body(h_ref, s_ref, d_ref, ws_ref, wn_ref, b_ref, o_ref):
    hn = _neigh(s_ref, d_ref)
    z = _mm(h_ref[...], ws_ref[...]) + _mm(hn, wn_ref[...]) + b_ref[...]
    o_ref[...] = _maxk_rows(z)


def _tc3_body(h_ref, s_ref, d_ref, ws_ref, wn_ref, b_ref, wo_ref, bo_ref,
              o_ref):
    hn = _neigh(s_ref, d_ref)
    z = _mm(h_ref[...], ws_ref[...]) + _mm(hn, wn_ref[...]) + b_ref[...]
    o_ref[...] = _mm(z, wo_ref[...]) + bo_ref[...]


_ROWS = pl.BlockSpec((_R, F), lambda i: (i, 0))
_WFULL = pl.BlockSpec((F, F), lambda i: (0, 0))
_BFULL = pl.BlockSpec((1, F), lambda i: (0, 0))
_SPART = pl.BlockSpec((_NC, _R, F), lambda i: (0, i, 0))
_DPART = pl.BlockSpec((_NC, _R, 1), lambda i: (0, i, 0))
_OSHAPE = jax.ShapeDtypeStruct((N, F), jnp.float32)


def kernel(x, edge_index, W_in, b_in, Wself0, Wneigh0, b0,
           Wself1, Wneigh1, b1, W_out, b_out):
    src = edge_index[0].astype(jnp.int32)
    dst = edge_index[1].astype(jnp.int32)

    h0 = pl.pallas_call(
        _tc1_body, grid=(_NB,),
        in_specs=[_ROWS, _WFULL, _BFULL],
        out_specs=_ROWS, out_shape=_OSHAPE,
    )(x, W_in, b_in.reshape(1, F))

    s1, d1 = _segsum(h0, src, dst)
    d1r = d1.reshape(_NC, NPAD, 1)

    h1 = pl.pallas_call(
        _tc2_body, grid=(_NB,),
        in_specs=[_ROWS, _SPART, _DPART, _WFULL, _WFULL, _BFULL],
        out_specs=_ROWS, out_shape=_OSHAPE,
    )(h0, s1, d1r, Wself0, Wneigh0, b0.reshape(1, F))

    s2, _ = _segsum(h1, src, dst)

    out = pl.pallas_call(
        _tc3_body, grid=(_NB,),
        in_specs=[_ROWS, _SPART, _DPART, _WFULL, _WFULL, _BFULL, _WFULL,
                  _BFULL],
        out_specs=_ROWS, out_shape=_OSHAPE,
    )(h1, s2, d1r, Wself1, Wneigh1, b1.reshape(1, F), W_out,
      b_out.reshape(1, F))

    return out


# speedup vs baseline: 15.8139x; 1.5434x over previous
import jax
import jax.numpy as jnp
import numpy as np
from jax import lax
from jax.experimental import pallas as pl
from jax.experimental.pallas import tpu as pltpu
from jax.experimental.pallas import tpu_sc as plsc

N = 10000
E = 320000
F = 128
K = 32

NPAD = 10240
_NC = 2
_NS = 16
_NW = _NC * _NS
_CHUNK = 128
_NBLK = E // _CHUNK
_BLK_PER_W = -(-_NBLK // _NW)
_KMAX = 80
_RPT = NPAD // _NS
_R = 1000
_NB = N // _R


def _seg_body(h_hbm, ei_hbm, s_hbm, deg_hbm,
              acc_sh, deg_sh, ones_v,
              idx0, idx1, idx2, idx3, idx4, idx5, idx6, idx7,
              rows0, rows1,
              si0, si1, si2, si3, si4, si5, si6, si7,
              sg0, sg1, ss0, ss1):
    cid = lax.axis_index("c")
    sid = lax.axis_index("s")
    wid = sid * _NC + cid
    idx = (idx0, idx1, idx2, idx3, idx4, idx5, idx6, idx7)
    rows = (rows0, rows1)
    si = (si0, si1, si2, si3, si4, si5, si6, si7)
    sg = (sg0, sg1)
    ss = (ss0, ss1)
    nw = (np.int32(_NBLK - 1) - wid) // _NW + 1

    def zfill(i, carry):
        r = i // 8
        c = (i % 8) * 16
        rows0[r, pl.ds(c, 16)] = jnp.zeros((16,), jnp.float32)
        return carry
    lax.fori_loop(0, _CHUNK * 8, zfill, 0)

    def ofill(i, carry):
        ones_v[pl.ds(i * 16, 16)] = jnp.ones((16,), jnp.float32)
        return carry
    lax.fori_loop(0, 8, ofill, 0)

    base_row = pl.multiple_of(sid * _RPT, 128)

    def zslab(k, carry):
        off = pl.multiple_of(base_row + k * 128, 128)
        pltpu.sync_copy(rows0, acc_sh.at[pl.ds(off, 128)])
        pltpu.sync_copy(rows0.at[0], deg_sh.at[pl.ds(off, 128)])
        return carry
    lax.fori_loop(0, _RPT // 128, zslab, 0)

    plsc.subcore_barrier()

    def idx_copy(k, b):
        base = pl.multiple_of((wid + _NW * k) * _CHUNK, _CHUNK)
        return pltpu.make_async_copy(ei_hbm.at[:, pl.ds(base, _CHUNK)],
                                     idx[b], si[b])

    def gather(b8, b2):
        return pltpu.make_async_copy(h_hbm.at[idx[b8].at[0]], rows[b2],
                                     sg[b2])

    def scat_start(b8, b2):
        pltpu.async_copy(rows[b2], acc_sh.at[idx[b8].at[1]], ss[b2],
                         add=True)
        pltpu.async_copy(ones_v, deg_sh.at[idx[b8].at[1]], ss[b2], add=True)

    def scat_wait(b8, b2):
        pltpu.make_async_copy(rows[b2], acc_sh.at[idx[b8].at[1]],
                              ss[b2]).wait()
        pltpu.make_async_copy(ones_v, deg_sh.at[idx[b8].at[1]],
                              ss[b2]).wait()

    for b in range(6):
        @pl.when(b < nw)
        def _(b=b):
            idx_copy(b, b).start()

    @pl.when(0 < nw)
    def _():
        idx_copy(0, 0).wait()
        gather(0, 0).start()

    def substep(k, u, drain):
        u2 = u % 2

        @pl.when(k < nw)
        def _():
            gather(u, u2).wait()
            scat_start(u, u2)

        if drain:
            @pl.when(k - 1 < nw)
            def _():
                scat_wait((u - 1) % 8, (u - 1) % 2)

        @pl.when(k + 1 < nw)
        def _():
            idx_copy(k + 1, (u + 1) % 8).wait()
            gather((u + 1) % 8, (u + 1) % 2).start()

        @pl.when(k + 6 < nw)
        def _():
            idx_copy(k + 6, (u + 6) % 8).start()

    for u in range(8):
        substep(u, u, u >= 1)

    def step(j, carry):
        for u in range(8):
            substep(j * 8 + u, u, True)
        return carry
    lax.fori_loop(1, _KMAX // 8, step, 0)

    @pl.when(_KMAX - 1 < nw)
    def _():
        scat_wait((_KMAX - 1) % 8, (_KMAX - 1) % 2)

    plsc.subcore_barrier()

    pltpu.sync_copy(acc_sh.at[pl.ds(base_row, _RPT)],
                    s_hbm.at[cid, pl.ds(base_row, _RPT)])
    pltpu.sync_copy(deg_sh.at[pl.ds(base_row, _RPT)],
                    deg_hbm.at[cid, pl.ds(base_row, _RPT)])


def _segsum(h, edge_index):
    call = pl.kernel(
        _seg_body,
        out_type=[
            jax.ShapeDtypeStruct((_NC, NPAD, F), jnp.float32),
            jax.ShapeDtypeStruct((_NC, NPAD), jnp.float32),
        ],
        mesh=plsc.VectorSubcoreMesh(core_axis_name="c", subcore_axis_name="s"),
        scratch_types=(
            [pltpu.VMEM_SHARED((NPAD, F), jnp.float32),
             pltpu.VMEM_SHARED((NPAD,), jnp.float32),
             pltpu.VMEM((_CHUNK,), jnp.float32)]
            + [pltpu.VMEM((2, _CHUNK), jnp.int32) for _ in range(8)]
            + [pltpu.VMEM((_CHUNK, F), jnp.float32) for _ in range(2)]
            + [pltpu.SemaphoreType.DMA for _ in range(12)]
        ),
    )
    return call(h, edge_index)


_MIN32 = np.int32(-(2 ** 31))


def _maxk_rows(x):
    key = lax.bitcast_convert_type(x, jnp.int32)
    key = key ^ ((key >> 31) & np.int32(0x7FFFFFFF))
    t = jnp.zeros((x.shape[0], 1), jnp.int32)
    for i in range(31, -1, -1):
        bit = _MIN32 if i == 31 else np.int32(1 << i)
        cand = t | bit
        cnt = jnp.sum((key >= (cand ^ _MIN32)).astype(jnp.int32),
                      axis=1, keepdims=True)
        t = jnp.where(cnt >= K, cand, t)
    thr = t ^ _MIN32
    gt = key > thr
    eq = key == thr
    ngt = jnp.sum(gt.astype(jnp.int32), axis=1, keepdims=True)
    quota = (K - ngt).astype(jnp.float32)
    ii = lax.broadcasted_iota(jnp.int32, (F, F), 0)
    jj = lax.broadcasted_iota(jnp.int32, (F, F), 1)
    upper = (ii <= jj).astype(jnp.float32)
    run = lax.dot_general(eq.astype(jnp.float32), upper,
                          (((1,), (0,)), ((), ())),
                          preferred_element_type=jnp.float32)
    keep = gt | (eq & (run <= quota))
    return jnp.where(keep, x, jnp.zeros_like(x))


def _mm(a, w):
    return lax.dot_general(a, w, (((1,), (1,)), ((), ())),
                           preferred_element_type=jnp.float32)


def _tc1_body(x_ref, w_ref, b_ref, o_ref):
    h = _mm(x_ref[...], w_ref[...]) + b_ref[...]
    o_ref[...] = _maxk_rows(h)


def _neigh(s_ref, d_ref):
    d = d_ref[0] + d_ref[1]
    return (s_ref[0] + s_ref[1]) * (1.0 / jnp.maximum(d, 1.0))


def _tc2_body(h_ref, s_ref, d_ref, ws_ref, wn_ref, b_ref, o_ref):
    hn = _neigh(s_ref, d_ref)
    z = _mm(h_ref[...], ws_ref[...]) + _mm(hn, wn_ref[...]) + b_ref[...]
    o_ref[...] = _maxk_rows(z)


def _tc3_body(h_ref, s_ref, d_ref, ws_ref, wn_ref, b_ref, wo_ref, bo_ref,
              o_ref):
    hn = _neigh(s_ref, d_ref)
    z = _mm(h_ref[...], ws_ref[...]) + _mm(hn, wn_ref[...]) + b_ref[...]
    o_ref[...] = _mm(z, wo_ref[...]) + bo_ref[...]


_ROWS = pl.BlockSpec((_R, F), lambda i: (i, 0))
_WFULL = pl.BlockSpec((F, F), lambda i: (0, 0))
_BFULL = pl.BlockSpec((1, F), lambda i: (0, 0))
_SPART = pl.BlockSpec((_NC, _R, F), lambda i: (0, i, 0))
_DPART = pl.BlockSpec((_NC, _R, 1), lambda i: (0, i, 0))
_OSHAPE = jax.ShapeDtypeStruct((N, F), jnp.float32)


def kernel(x, edge_index, W_in, b_in, Wself0, Wneigh0, b0,
           Wself1, Wneigh1, b1, W_out, b_out):
    ei = edge_index.astype(jnp.int32)

    h0 = pl.pallas_call(
        _tc1_body, grid=(_NB,),
        in_specs=[_ROWS, _WFULL, _BFULL],
        out_specs=_ROWS, out_shape=_OSHAPE,
    )(x, W_in, b_in.reshape(1, F))

    s1, d1 = _segsum(h0, ei)
    d1r = d1.reshape(_NC, NPAD, 1)

    h1 = pl.pallas_call(
        _tc2_body, grid=(_NB,),
        in_specs=[_ROWS, _SPART, _DPART, _WFULL, _WFULL, _BFULL],
        out_specs=_ROWS, out_shape=_OSHAPE,
    )(h0, s1, d1r, Wself0, Wneigh0, b0.reshape(1, F))

    s2, _ = _segsum(h1, ei)

    out = pl.pallas_call(
        _tc3_body, grid=(_NB,),
        in_specs=[_ROWS, _SPART, _DPART, _WFULL, _WFULL, _BFULL, _WFULL,
                  _BFULL],
        out_specs=_ROWS, out_shape=_OSHAPE,
    )(h1, s2, d1r, Wself1, Wneigh1, b1.reshape(1, F), W_out,
      b_out.reshape(1, F))

    return out


# speedup vs baseline: 17.4545x; 1.1037x over previous
import jax
import jax.numpy as jnp
import numpy as np
from jax import lax
from jax.experimental import pallas as pl
from jax.experimental.pallas import tpu as pltpu
from jax.experimental.pallas import tpu_sc as plsc

N = 10000
E = 320000
F = 128
K = 32

NPAD = 10240
_NC = 2
_NS = 16
_NW = _NC * _NS
_CHUNK = 128
_NBLK = E // _CHUNK
_BLK_PER_W = -(-_NBLK // _NW)
_KMAX = 80
_RPT = NPAD // _NS
_R = 1000
_NB = N // _R


def _seg_body(h_hbm, ei_hbm, s_hbm, deg_hbm,
              acc_sh, deg_sh, ones_v,
              idx0, idx1, idx2, idx3, idx4, idx5, idx6, idx7,
              rows0, rows1,
              si0, si1, si2, si3, si4, si5, si6, si7,
              sg0, sg1, ss0, ss1):
    cid = lax.axis_index("c")
    sid = lax.axis_index("s")
    wid = sid * _NC + cid
    idx = (idx0, idx1, idx2, idx3, idx4, idx5, idx6, idx7)
    rows = (rows0, rows1)
    si = (si0, si1, si2, si3, si4, si5, si6, si7)
    sg = (sg0, sg1)
    ss = (ss0, ss1)
    nw = (np.int32(_NBLK - 1) - wid) // _NW + 1

    def zfill(i, carry):
        r = i // 8
        c = (i % 8) * 16
        rows0[r, pl.ds(c, 16)] = jnp.zeros((16,), jnp.float32)
        return carry
    lax.fori_loop(0, _CHUNK * 8, zfill, 0)

    def ofill(i, carry):
        ones_v[pl.ds(i * 16, 16)] = jnp.ones((16,), jnp.float32)
        return carry
    lax.fori_loop(0, 8, ofill, 0)

    base_row = pl.multiple_of(sid * _RPT, 128)

    def zslab(k, carry):
        off = pl.multiple_of(base_row + k * 128, 128)
        pltpu.sync_copy(rows0, acc_sh.at[pl.ds(off, 128)])
        pltpu.sync_copy(rows0.at[0], deg_sh.at[pl.ds(off, 128)])
        return carry
    lax.fori_loop(0, _RPT // 128, zslab, 0)

    plsc.subcore_barrier()

    def idx_copy(k, b):
        base = pl.multiple_of((wid + _NW * k) * _CHUNK, _CHUNK)
        return pltpu.make_async_copy(ei_hbm.at[:, pl.ds(base, _CHUNK)],
                                     idx[b], si[b])

    def gather(b8, b2):
        return pltpu.make_async_copy(h_hbm.at[idx[b8].at[0]], rows[b2],
                                     sg[b2])

    def scat_start(b8, b2):
        pltpu.async_copy(rows[b2], acc_sh.at[idx[b8].at[1]], ss[b2],
                         add=True)
        pltpu.async_copy(ones_v, deg_sh.at[idx[b8].at[1]], ss[b2], add=True)

    def scat_wait(b8, b2):
        pltpu.make_async_copy(rows[b2], acc_sh.at[idx[b8].at[1]],
                              ss[b2]).wait()
        pltpu.make_async_copy(ones_v, deg_sh.at[idx[b8].at[1]],
                              ss[b2]).wait()

    for b in range(6):
        @pl.when(b < nw)
        def _(b=b):
            idx_copy(b, b).start()

    @pl.when(0 < nw)
    def _():
        idx_copy(0, 0).wait()
        gather(0, 0).start()

    def substep(k, u, drain):
        u2 = u % 2

        @pl.when(k < nw)
        def _():
            gather(u, u2).wait()
            scat_start(u, u2)

        if drain:
            @pl.when(k - 1 < nw)
            def _():
                scat_wait((u - 1) % 8, (u - 1) % 2)

        @pl.when(k + 1 < nw)
        def _():
            idx_copy(k + 1, (u + 1) % 8).wait()
            gather((u + 1) % 8, (u + 1) % 2).start()

        @pl.when(k + 6 < nw)
        def _():
            idx_copy(k + 6, (u + 6) % 8).start()

    for u in range(8):
        substep(u, u, u >= 1)

    def step(j, carry):
        for u in range(8):
            substep(j * 8 + u, u, True)
        return carry
    lax.fori_loop(1, _KMAX // 8, step, 0)

    @pl.when(_KMAX - 1 < nw)
    def _():
        scat_wait((_KMAX - 1) % 8, (_KMAX - 1) % 2)

    plsc.subcore_barrier()

    pltpu.sync_copy(acc_sh.at[pl.ds(base_row, _RPT)],
                    s_hbm.at[cid, pl.ds(base_row, _RPT)])
    pltpu.sync_copy(deg_sh.at[pl.ds(base_row, _RPT)],
                    deg_hbm.at[cid, pl.ds(base_row, _RPT)])


def _segsum(h, edge_index):
    call = pl.kernel(
        _seg_body,
        out_type=[
            jax.ShapeDtypeStruct((_NC, NPAD, F), jnp.float32),
            jax.ShapeDtypeStruct((_NC, NPAD), jnp.float32),
        ],
        mesh=plsc.VectorSubcoreMesh(core_axis_name="c", subcore_axis_name="s"),
        scratch_types=(
            [pltpu.VMEM_SHARED((NPAD, F), jnp.float32),
             pltpu.VMEM_SHARED((NPAD,), jnp.float32),
             pltpu.VMEM((_CHUNK,), jnp.float32)]
            + [pltpu.VMEM((2, _CHUNK), jnp.int32) for _ in range(8)]
            + [pltpu.VMEM((_CHUNK, F), jnp.float32) for _ in range(2)]
            + [pltpu.SemaphoreType.DMA for _ in range(12)]
        ),
    )
    return call(h, edge_index)


_MIN32 = np.int32(-(2 ** 31))


def _maxk_rows(x):
    key = lax.bitcast_convert_type(x, jnp.int32)
    key = key ^ ((key >> 31) & np.int32(0x7FFFFFFF))
    one = jnp.ones_like(x[:1])
    ones_col = lax.broadcast_in_dim(jnp.float32(1.0), (F, 1), ())

    def _count(mask):
        return lax.dot_general(jnp.where(mask, one, 0.0 * one), ones_col,
                               (((1,), (0,)), ((), ())),
                               preferred_element_type=jnp.float32)

    t = jnp.zeros((x.shape[0], 1), jnp.int32)
    for i in range(31, -1, -1):
        bit = _MIN32 if i == 31 else np.int32(1 << i)
        cand = t | bit
        cnt = _count(key >= (cand ^ _MIN32))
        t = jnp.where(cnt >= float(K), cand, t)
    thr = t ^ _MIN32
    gt = key > thr
    eq = key == thr
    quota = float(K) - _count(gt)
    ii = lax.broadcasted_iota(jnp.int32, (F, F), 0)
    jj = lax.broadcasted_iota(jnp.int32, (F, F), 1)
    upper = (ii <= jj).astype(jnp.float32)
    run = lax.dot_general(jnp.where(eq, one, 0.0 * one), upper,
                          (((1,), (0,)), ((), ())),
                          preferred_element_type=jnp.float32)
    keep = gt | (eq & (run <= quota))
    return jnp.where(keep, x, jnp.zeros_like(x))


def _mm(a, w):
    return lax.dot_general(a, w, (((1,), (1,)), ((), ())),
                           preferred_element_type=jnp.float32)


def _tc1_body(x_ref, w_ref, b_ref, o_ref):
    h = _mm(x_ref[...], w_ref[...]) + b_ref[...]
    o_ref[...] = _maxk_rows(h)


def _neigh(s_ref, d_ref):
    d = d_ref[0] + d_ref[1]
    return (s_ref[0] + s_ref[1]) * (1.0 / jnp.maximum(d, 1.0))


def _tc2_body(h_ref, s_ref, d_ref, ws_ref, wn_ref, b_ref, o_ref):
    hn = _neigh(s_ref, d_ref)
    z = _mm(h_ref[...], ws_ref[...]) + _mm(hn, wn_ref[...]) + b_ref[...]
    o_ref[...] = _maxk_rows(z)


def _tc3_body(h_ref, s_ref, d_ref, ws_ref, wn_ref, b_ref, wo_ref, bo_ref,
              o_ref):
    hn = _neigh(s_ref, d_ref)
    z = _mm(h_ref[...], ws_ref[...]) + _mm(hn, wn_ref[...]) + b_ref[...]
    o_ref[...] = _mm(z, wo_ref[...]) + bo_ref[...]


_ROWS = pl.BlockSpec((_R, F), lambda i: (i, 0))
_WFULL = pl.BlockSpec((F, F), lambda i: (0, 0))
_BFULL = pl.BlockSpec((1, F), lambda i: (0, 0))
_SPART = pl.BlockSpec((_NC, _R, F), lambda i: (0, i, 0))
_DPART = pl.BlockSpec((_NC, _R, 1), lambda i: (0, i, 0))
_OSHAPE = jax.ShapeDtypeStruct((N, F), jnp.float32)


def kernel(x, edge_index, W_in, b_in, Wself0, Wneigh0, b0,
           Wself1, Wneigh1, b1, W_out, b_out):
    ei = edge_index.astype(jnp.int32)

    h0 = pl.pallas_call(
        _tc1_body, grid=(_NB,),
        in_specs=[_ROWS, _WFULL, _BFULL],
        out_specs=_ROWS, out_shape=_OSHAPE,
    )(x, W_in, b_in.reshape(1, F))

    s1, d1 = _segsum(h0, ei)
    d1r = d1.reshape(_NC, NPAD, 1)

    h1 = pl.pallas_call(
        _tc2_body, grid=(_NB,),
        in_specs=[_ROWS, _SPART, _DPART, _WFULL, _WFULL, _BFULL],
        out_specs=_ROWS, out_shape=_OSHAPE,
    )(h0, s1, d1r, Wself0, Wneigh0, b0.reshape(1, F))

    s2, _ = _segsum(h1, ei)

    out = pl.pallas_call(
        _tc3_body, grid=(_NB,),
        in_specs=[_ROWS, _SPART, _DPART, _WFULL, _WFULL, _BFULL, _WFULL,
                  _BFULL],
        out_specs=_ROWS, out_shape=_OSHAPE,
    )(h1, s2, d1r, Wself1, Wneigh1, b1.reshape(1, F), W_out,
      b_out.reshape(1, F))

    return out


# speedup vs baseline: 19.3882x; 1.1108x over previous
import jax
import jax.numpy as jnp
import numpy as np
from jax import lax
from jax.experimental import pallas as pl
from jax.experimental.pallas import tpu as pltpu
from jax.experimental.pallas import tpu_sc as plsc

N = 10000
E = 320000
F = 128
K = 32

NPAD = 10240
_NC = 2
_NS = 16
_NW = _NC * _NS
_CHUNK = 128
_NBLK = E // _CHUNK
_BLK_PER_W = -(-_NBLK // _NW)
_KMAX = 80
_RPT = NPAD // _NS
_R = 2000
_NB = N // _R


def _seg_body(h_hbm, ei_hbm, s_hbm, deg_hbm,
              acc_sh, deg_sh, ones_v,
              idx0, idx1, idx2, idx3, idx4, idx5, idx6, idx7,
              rows0, rows1,
              si0, si1, si2, si3, si4, si5, si6, si7,
              sg0, sg1, ss0, ss1):
    cid = lax.axis_index("c")
    sid = lax.axis_index("s")
    wid = sid * _NC + cid
    idx = (idx0, idx1, idx2, idx3, idx4, idx5, idx6, idx7)
    rows = (rows0, rows1)
    si = (si0, si1, si2, si3, si4, si5, si6, si7)
    sg = (sg0, sg1)
    ss = (ss0, ss1)
    nw = (np.int32(_NBLK - 1) - wid) // _NW + 1

    def zfill(i, carry):
        r = i // 8
        c = (i % 8) * 16
        rows0[r, pl.ds(c, 16)] = jnp.zeros((16,), jnp.float32)
        return carry
    lax.fori_loop(0, _CHUNK * 8, zfill, 0)

    def ofill(i, carry):
        ones_v[pl.ds(i * 16, 16)] = jnp.ones((16,), jnp.float32)
        return carry
    lax.fori_loop(0, 8, ofill, 0)

    base_row = pl.multiple_of(sid * _RPT, 128)

    def zslab(k, carry):
        off = pl.multiple_of(base_row + k * 128, 128)
        pltpu.sync_copy(rows0, acc_sh.at[pl.ds(off, 128)])
        pltpu.sync_copy(rows0.at[0], deg_sh.at[pl.ds(off, 128)])
        return carry
    lax.fori_loop(0, _RPT // 128, zslab, 0)

    plsc.subcore_barrier()

    def idx_copy(k, b):
        base = pl.multiple_of((wid + _NW * k) * _CHUNK, _CHUNK)
        return pltpu.make_async_copy(ei_hbm.at[:, pl.ds(base, _CHUNK)],
                                     idx[b], si[b])

    def gather(b8, b2):
        return pltpu.make_async_copy(h_hbm.at[idx[b8].at[0]], rows[b2],
                                     sg[b2])

    def scat_start(b8, b2):
        pltpu.async_copy(rows[b2], acc_sh.at[idx[b8].at[1]], ss[b2],
                         add=True)
        pltpu.async_copy(ones_v, deg_sh.at[idx[b8].at[1]], ss[b2], add=True)

    def scat_wait(b8, b2):
        pltpu.make_async_copy(rows[b2], acc_sh.at[idx[b8].at[1]],
                              ss[b2]).wait()
        pltpu.make_async_copy(ones_v, deg_sh.at[idx[b8].at[1]],
                              ss[b2]).wait()

    for b in range(6):
        @pl.when(b < nw)
        def _(b=b):
            idx_copy(b, b).start()

    @pl.when(0 < nw)
    def _():
        idx_copy(0, 0).wait()
        gather(0, 0).start()

    def substep(k, u, drain):
        u2 = u % 2

        @pl.when(k < nw)
        def _():
            gather(u, u2).wait()
            scat_start(u, u2)

        if drain:
            @pl.when(k - 1 < nw)
            def _():
                scat_wait((u - 1) % 8, (u - 1) % 2)

        @pl.when(k + 1 < nw)
        def _():
            idx_copy(k + 1, (u + 1) % 8).wait()
            gather((u + 1) % 8, (u + 1) % 2).start()

        @pl.when(k + 6 < nw)
        def _():
            idx_copy(k + 6, (u + 6) % 8).start()

    for u in range(8):
        substep(u, u, u >= 1)

    def step(j, carry):
        for u in range(8):
            substep(j * 8 + u, u, True)
        return carry
    lax.fori_loop(1, _KMAX // 8, step, 0)

    @pl.when(_KMAX - 1 < nw)
    def _():
        scat_wait((_KMAX - 1) % 8, (_KMAX - 1) % 2)

    plsc.subcore_barrier()

    pltpu.sync_copy(acc_sh.at[pl.ds(base_row, _RPT)],
                    s_hbm.at[cid, pl.ds(base_row, _RPT)])
    pltpu.sync_copy(deg_sh.at[pl.ds(base_row, _RPT)],
                    deg_hbm.at[cid, pl.ds(base_row, _RPT)])


def _segsum(h, edge_index):
    call = pl.kernel(
        _seg_body,
        out_type=[
            jax.ShapeDtypeStruct((_NC, NPAD, F), jnp.float32),
            jax.ShapeDtypeStruct((_NC, NPAD), jnp.float32),
        ],
        mesh=plsc.VectorSubcoreMesh(core_axis_name="c", subcore_axis_name="s"),
        scratch_types=(
            [pltpu.VMEM_SHARED((NPAD, F), jnp.float32),
             pltpu.VMEM_SHARED((NPAD,), jnp.float32),
             pltpu.VMEM((_CHUNK,), jnp.float32)]
            + [pltpu.VMEM((2, _CHUNK), jnp.int32) for _ in range(8)]
            + [pltpu.VMEM((_CHUNK, F), jnp.float32) for _ in range(2)]
            + [pltpu.SemaphoreType.DMA for _ in range(12)]
        ),
    )
    return call(h, edge_index)


_MIN32 = np.int32(-(2 ** 31))


def _maxk_rows(x):
    key = lax.bitcast_convert_type(x, jnp.int32)
    key = key ^ ((key >> 31) & np.int32(0x7FFFFFFF))
    one = jnp.ones_like(x[:1])
    ones_col = lax.broadcast_in_dim(jnp.float32(1.0), (F, 1), ())

    def _count(mask):
        return lax.dot_general(jnp.where(mask, one, 0.0 * one), ones_col,
                               (((1,), (0,)), ((), ())),
                               preferred_element_type=jnp.float32)

    t = jnp.zeros((x.shape[0], 1), jnp.int32)
    for i in range(31, -1, -1):
        bit = _MIN32 if i == 31 else np.int32(1 << i)
        cand = t | bit
        cnt = _count(key >= (cand ^ _MIN32))
        t = jnp.where(cnt >= float(K), cand, t)
    thr = t ^ _MIN32
    gt = key > thr
    eq = key == thr
    quota = float(K) - _count(gt)
    ii = lax.broadcasted_iota(jnp.int32, (F, F), 0)
    jj = lax.broadcasted_iota(jnp.int32, (F, F), 1)
    upper = (ii <= jj).astype(jnp.float32)
    run = lax.dot_general(jnp.where(eq, one, 0.0 * one), upper,
                          (((1,), (0,)), ((), ())),
                          preferred_element_type=jnp.float32)
    keep = gt | (eq & (run <= quota))
    return jnp.where(keep, x, jnp.zeros_like(x))


def _mm(a, w):
    return lax.dot_general(a, w, (((1,), (1,)), ((), ())),
                           preferred_element_type=jnp.float32)


def _tc1_body(x_ref, w_ref, b_ref, o_ref):
    h = _mm(x_ref[...], w_ref[...]) + b_ref[...]
    o_ref[...] = _maxk_rows(h)


def _neigh(s_ref, d_ref):
    d = d_ref[0] + d_ref[1]
    return (s_ref[0] + s_ref[1]) * (1.0 / jnp.maximum(d, 1.0))


def _tc2_body(h_ref, s_ref, d_ref, ws_ref, wn_ref, b_ref, o_ref):
    hn = _neigh(s_ref, d_ref)
    z = _mm(h_ref[...], ws_ref[...]) + _mm(hn, wn_ref[...]) + b_ref[...]
    o_ref[...] = _maxk_rows(z)


def _tc3_body(h_ref, s_ref, d_ref, ws_ref, wn_ref, b_ref, wo_ref, bo_ref,
              o_ref):
    hn = _neigh(s_ref, d_ref)
    z = _mm(h_ref[...], ws_ref[...]) + _mm(hn, wn_ref[...]) + b_ref[...]
    o_ref[...] = _mm(z, wo_ref[...]) + bo_ref[...]


_ROWS = pl.BlockSpec((_R, F), lambda i: (i, 0))
_WFULL = pl.BlockSpec((F, F), lambda i: (0, 0))
_BFULL = pl.BlockSpec((1, F), lambda i: (0, 0))
_SPART = pl.BlockSpec((_NC, _R, F), lambda i: (0, i, 0))
_DPART = pl.BlockSpec((_NC, _R, 1), lambda i: (0, i, 0))
_OSHAPE = jax.ShapeDtypeStruct((N, F), jnp.float32)


def kernel(x, edge_index, W_in, b_in, Wself0, Wneigh0, b0,
           Wself1, Wneigh1, b1, W_out, b_out):
    ei = edge_index.astype(jnp.int32)

    h0 = pl.pallas_call(
        _tc1_body, grid=(_NB,),
        in_specs=[_ROWS, _WFULL, _BFULL],
        out_specs=_ROWS, out_shape=_OSHAPE,
    )(x, W_in, b_in.reshape(1, F))

    s1, d1 = _segsum(h0, ei)
    d1r = d1.reshape(_NC, NPAD, 1)

    h1 = pl.pallas_call(
        _tc2_body, grid=(_NB,),
        in_specs=[_ROWS, _SPART, _DPART, _WFULL, _WFULL, _BFULL],
        out_specs=_ROWS, out_shape=_OSHAPE,
    )(h0, s1, d1r, Wself0, Wneigh0, b0.reshape(1, F))

    s2, _ = _segsum(h1, ei)

    out = pl.pallas_call(
        _tc3_body, grid=(_NB,),
        in_specs=[_ROWS, _SPART, _DPART, _WFULL, _WFULL, _BFULL, _WFULL,
                  _BFULL],
        out_specs=_ROWS, out_shape=_OSHAPE,
    )(h1, s2, d1r, Wself1, Wneigh1, b1.reshape(1, F), W_out,
      b_out.reshape(1, F))

    return out


# speedup vs baseline: 19.5202x; 1.0068x over previous
import jax
import jax.numpy as jnp
import numpy as np
from jax import lax
from jax.experimental import pallas as pl
from jax.experimental.pallas import tpu as pltpu
from jax.experimental.pallas import tpu_sc as plsc

N = 10000
E = 320000
F = 128
K = 32

NPAD = 10240
_NC = 2
_NS = 16
_NW = _NC * _NS
_CHUNK = 128
_NBLK = E // _CHUNK
_BLK_PER_W = -(-_NBLK // _NW)
_KMAX = 80
_RPT = NPAD // _NS
_R = 2000
_NB = N // _R


def _seg_body_deg(h_hbm, ei_hbm, s_hbm, deg_hbm, *rest):
    _seg_body(True, h_hbm, ei_hbm, s_hbm, deg_hbm, *rest)


def _seg_body_nodeg(h_hbm, ei_hbm, s_hbm, *rest):
    _seg_body(False, h_hbm, ei_hbm, s_hbm, None, *rest)


def _seg_body(with_deg, h_hbm, ei_hbm, s_hbm, deg_hbm,
              acc_sh, deg_sh, ones_v,
              idx0, idx1, idx2, idx3, idx4, idx5, idx6, idx7,
              rows0, rows1,
              si0, si1, si2, si3, si4, si5, si6, si7,
              sg0, sg1, ss0, ss1):
    cid = lax.axis_index("c")
    sid = lax.axis_index("s")
    wid = sid * _NC + cid
    idx = (idx0, idx1, idx2, idx3, idx4, idx5, idx6, idx7)
    rows = (rows0, rows1)
    si = (si0, si1, si2, si3, si4, si5, si6, si7)
    sg = (sg0, sg1)
    ss = (ss0, ss1)
    nw = (np.int32(_NBLK - 1) - wid) // _NW + 1

    def zfill(i, carry):
        r = i // 8
        c = (i % 8) * 16
        rows0[r, pl.ds(c, 16)] = jnp.zeros((16,), jnp.float32)
        return carry
    lax.fori_loop(0, _CHUNK * 8, zfill, 0)

    def ofill(i, carry):
        ones_v[pl.ds(i * 16, 16)] = jnp.ones((16,), jnp.float32)
        return carry
    lax.fori_loop(0, 8, ofill, 0)

    base_row = pl.multiple_of(sid * _RPT, 128)

    def zslab(k, carry):
        off = pl.multiple_of(base_row + k * 128, 128)
        pltpu.sync_copy(rows0, acc_sh.at[pl.ds(off, 128)])
        if with_deg:
            pltpu.sync_copy(rows0.at[0], deg_sh.at[pl.ds(off, 128)])
        return carry
    lax.fori_loop(0, _RPT // 128, zslab, 0)

    plsc.subcore_barrier()

    def idx_copy(k, b):
        base = pl.multiple_of((wid + _NW * k) * _CHUNK, _CHUNK)
        return pltpu.make_async_copy(ei_hbm.at[:, pl.ds(base, _CHUNK)],
                                     idx[b], si[b])

    def gather(b8, b2):
        return pltpu.make_async_copy(h_hbm.at[idx[b8].at[0]], rows[b2],
                                     sg[b2])

    def scat_start(b8, b2):
        pltpu.async_copy(rows[b2], acc_sh.at[idx[b8].at[1]], ss[b2],
                         add=True)
        if with_deg:
            pltpu.async_copy(ones_v, deg_sh.at[idx[b8].at[1]], ss[b2],
                             add=True)

    def scat_wait(b8, b2):
        pltpu.make_async_copy(rows[b2], acc_sh.at[idx[b8].at[1]],
                              ss[b2]).wait()
        if with_deg:
            pltpu.make_async_copy(ones_v, deg_sh.at[idx[b8].at[1]],
                                  ss[b2]).wait()

    for b in range(6):
        @pl.when(b < nw)
        def _(b=b):
            idx_copy(b, b).start()

    @pl.when(0 < nw)
    def _():
        idx_copy(0, 0).wait()
        gather(0, 0).start()

    def substep(k, u, drain):
        u2 = u % 2

        @pl.when(k < nw)
        def _():
            gather(u, u2).wait()
            scat_start(u, u2)

        if drain:
            @pl.when(k - 1 < nw)
            def _():
                scat_wait((u - 1) % 8, (u - 1) % 2)

        @pl.when(k + 1 < nw)
        def _():
            idx_copy(k + 1, (u + 1) % 8).wait()
            gather((u + 1) % 8, (u + 1) % 2).start()

        @pl.when(k + 6 < nw)
        def _():
            idx_copy(k + 6, (u + 6) % 8).start()

    for u in range(8):
        substep(u, u, u >= 1)

    def step(j, carry):
        for u in range(8):
            substep(j * 8 + u, u, True)
        return carry
    lax.fori_loop(1, _KMAX // 8, step, 0)

    @pl.when(_KMAX - 1 < nw)
    def _():
        scat_wait((_KMAX - 1) % 8, (_KMAX - 1) % 2)

    plsc.subcore_barrier()

    pltpu.sync_copy(acc_sh.at[pl.ds(base_row, _RPT)],
                    s_hbm.at[cid, pl.ds(base_row, _RPT)])
    if with_deg:
        pltpu.sync_copy(deg_sh.at[pl.ds(base_row, _RPT)],
                        deg_hbm.at[cid, pl.ds(base_row, _RPT)])


def _segsum(h, edge_index, with_deg=True):
    out_type = [jax.ShapeDtypeStruct((_NC, NPAD, F), jnp.float32)]
    if with_deg:
        out_type.append(jax.ShapeDtypeStruct((_NC, NPAD), jnp.float32))
    call = pl.kernel(
        _seg_body_deg if with_deg else _seg_body_nodeg,
        out_type=out_type,
        mesh=plsc.VectorSubcoreMesh(core_axis_name="c", subcore_axis_name="s"),
        scratch_types=(
            [pltpu.VMEM_SHARED((NPAD, F), jnp.float32),
             pltpu.VMEM_SHARED((NPAD,), jnp.float32),
             pltpu.VMEM((_CHUNK,), jnp.float32)]
            + [pltpu.VMEM((2, _CHUNK), jnp.int32) for _ in range(8)]
            + [pltpu.VMEM((_CHUNK, F), jnp.float32) for _ in range(2)]
            + [pltpu.SemaphoreType.DMA for _ in range(12)]
        ),
    )
    return call(h, edge_index)


_MIN32 = np.int32(-(2 ** 31))


def _maxk_rows(x):
    key = lax.bitcast_convert_type(x, jnp.int32)
    key = key ^ ((key >> 31) & np.int32(0x7FFFFFFF))
    one = jnp.ones_like(x[:1])
    ones_col = lax.broadcast_in_dim(jnp.float32(1.0), (F, 1), ())

    def _count(mask):
        return lax.dot_general(jnp.where(mask, one, 0.0 * one), ones_col,
                               (((1,), (0,)), ((), ())),
                               preferred_element_type=jnp.float32)

    t = jnp.zeros((x.shape[0], 1), jnp.int32)
    for i in range(31, -1, -1):
        bit = _MIN32 if i == 31 else np.int32(1 << i)
        cand = t | bit
        cnt = _count(key >= (cand ^ _MIN32))
        t = jnp.where(cnt >= float(K), cand, t)
    thr = t ^ _MIN32
    gt = key > thr
    eq = key == thr
    quota = float(K) - _count(gt)
    ii = lax.broadcasted_iota(jnp.int32, (F, F), 0)
    jj = lax.broadcasted_iota(jnp.int32, (F, F), 1)
    upper = (ii <= jj).astype(jnp.float32)
    run = lax.dot_general(jnp.where(eq, one, 0.0 * one), upper,
                          (((1,), (0,)), ((), ())),
                          preferred_element_type=jnp.float32)
    keep = gt | (eq & (run <= quota))
    return jnp.where(keep, x, jnp.zeros_like(x))


def _mm(a, w):
    return lax.dot_general(a, w, (((1,), (1,)), ((), ())),
                           preferred_element_type=jnp.float32)


def _tc1_body(x_ref, w_ref, b_ref, o_ref):
    h = _mm(x_ref[...], w_ref[...]) + b_ref[...]
    o_ref[...] = _maxk_rows(h)


def _neigh(s_ref, d_ref):
    d = d_ref[0] + d_ref[1]
    return (s_ref[0] + s_ref[1]) * (1.0 / jnp.maximum(d, 1.0))


def _tc2_body(h_ref, s_ref, d_ref, ws_ref, wn_ref, b_ref, o_ref):
    hn = _neigh(s_ref, d_ref)
    z = _mm(h_ref[...], ws_ref[...]) + _mm(hn, wn_ref[...]) + b_ref[...]
    o_ref[...] = _maxk_rows(z)


def _tc3_body(h_ref, s_ref, d_ref, ws_ref, wn_ref, b_ref, wo_ref, bo_ref,
              o_ref):
    hn = _neigh(s_ref, d_ref)
    z = _mm(h_ref[...], ws_ref[...]) + _mm(hn, wn_ref[...]) + b_ref[...]
    o_ref[...] = _mm(z, wo_ref[...]) + bo_ref[...]


_ROWS = pl.BlockSpec((_R, F), lambda i: (i, 0))
_WFULL = pl.BlockSpec((F, F), lambda i: (0, 0))
_BFULL = pl.BlockSpec((1, F), lambda i: (0, 0))
_SPART = pl.BlockSpec((_NC, _R, F), lambda i: (0, i, 0))
_DPART = pl.BlockSpec((_NC, _R, 1), lambda i: (0, i, 0))
_OSHAPE = jax.ShapeDtypeStruct((N, F), jnp.float32)


def kernel(x, edge_index, W_in, b_in, Wself0, Wneigh0, b0,
           Wself1, Wneigh1, b1, W_out, b_out):
    ei = edge_index.astype(jnp.int32)

    h0 = pl.pallas_call(
        _tc1_body, grid=(_NB,),
        in_specs=[_ROWS, _WFULL, _BFULL],
        out_specs=_ROWS, out_shape=_OSHAPE,
    )(x, W_in, b_in.reshape(1, F))

    s1, d1 = _segsum(h0, ei)
    d1r = d1.reshape(_NC, NPAD, 1)

    h1 = pl.pallas_call(
        _tc2_body, grid=(_NB,),
        in_specs=[_ROWS, _SPART, _DPART, _WFULL, _WFULL, _BFULL],
        out_specs=_ROWS, out_shape=_OSHAPE,
    )(h0, s1, d1r, Wself0, Wneigh0, b0.reshape(1, F))

    (s2,) = _segsum(h1, ei, with_deg=False)

    out = pl.pallas_call(
        _tc3_body, grid=(_NB,),
        in_specs=[_ROWS, _SPART, _DPART, _WFULL, _WFULL, _BFULL, _WFULL,
                  _BFULL],
        out_specs=_ROWS, out_shape=_OSHAPE,
    )(h1, s2, d1r, Wself1, Wneigh1, b1.reshape(1, F), W_out,
      b_out.reshape(1, F))

    return out
